# Initial kernel scaffold; baseline (speedup 1.0000x reference)
import jax
import jax.numpy as jnp
from jax.experimental import pallas as pl


def kernel(feature, flow_adj, flow_char_adj, item_id, category, PA_level, weight_emb, bias_emb, weight_character, a_attn, weight_flow):
    raise NotImplementedError("write your pallas kernel here")


# speedup vs baseline: 5.7724x; 5.7724x over previous
import functools

import jax
import jax.numpy as jnp
from jax.experimental import pallas as pl
from jax.experimental.pallas import tpu as pltpu
from jax.experimental.pallas import tpu_sc as plsc

_N = 10000
_DEG = 16
_M = 1024
_CDEG = 64
_B = 4096
_F = 256

_GW = 128


def _sc_mesh():
    return plsc.VectorSubcoreMesh(core_axis_name="core", subcore_axis_name="subcore")


def _gather_pipeline(table_hbm, idx_hbm, out_hbm, n_idx, row_w):
    def body(i_vmem, o_vmem):
        pltpu.sync_copy(table_hbm.at[i_vmem.at[0]], o_vmem)

    pltpu.emit_pipeline(
        body,
        grid=(n_idx // _GW,),
        in_specs=[pl.BlockSpec((1, _GW), lambda i: (0, i))],
        out_specs=[pl.BlockSpec((_GW, row_w), lambda i: (i, 0))],
        core_axis_name=("core", "subcore"),
        dimension_semantics=(pltpu.PARALLEL,),
    )(idx_hbm, out_hbm)


def _sc_gather_feat_nbrs(feature, flow_adj_pad, idx_a, idx_b):
    na = idx_a.shape[1]
    nb = idx_b.shape[1]

    @functools.partial(
        pl.kernel,
        mesh=_sc_mesh(),
        out_type=[
            jax.ShapeDtypeStruct((na, _F), jnp.float32),
            jax.ShapeDtypeStruct((nb, 128), jnp.int32),
        ],
    )
    def k(feat_hbm, fadj_hbm, ia_hbm, ib_hbm, r1_hbm, nb_hbm):
        _gather_pipeline(feat_hbm, ia_hbm, r1_hbm, na, _F)
        _gather_pipeline(fadj_hbm, ib_hbm, nb_hbm, nb, 128)

    return k(feature, flow_adj_pad, idx_a, idx_b)


def _sc_gather_rows(table, idx):
    n = idx.shape[1]
    w = table.shape[1]

    @functools.partial(
        pl.kernel,
        mesh=_sc_mesh(),
        out_type=jax.ShapeDtypeStruct((n, w), jnp.float32),
    )
    def k(tab_hbm, i_hbm, o_hbm):
        _gather_pipeline(tab_hbm, i_hbm, o_hbm, n, w)

    return k(table, idx)


def _sc_gather_uv(u_tab, v_tab, idx_u, idx_v):
    nu = idx_u.shape[1]
    nv = idx_v.shape[1]

    @functools.partial(
        pl.kernel,
        mesh=_sc_mesh(),
        out_type=[
            jax.ShapeDtypeStruct((nu, _F), jnp.float32),
            jax.ShapeDtypeStruct((nv, _F), jnp.float32),
        ],
    )
    def k(u_hbm, v_hbm, iu_hbm, iv_hbm, uu_hbm, vv_hbm):
        _gather_pipeline(u_hbm, iu_hbm, uu_hbm, nu, _F)
        _gather_pipeline(v_hbm, iv_hbm, vv_hbm, nv, _F)

    return k(u_tab, v_tab, idx_u, idx_v)


def _segsum_body(r_ref, o_ref):
    for j in range(8):
        o_ref[j : j + 1, :] = jnp.sum(
            r_ref[pl.ds(j * _CDEG, _CDEG), :], axis=0, keepdims=True
        )


def _tc_segsum(r_char):
    return pl.pallas_call(
        _segsum_body,
        grid=(_M // 8,),
        in_specs=[pl.BlockSpec((8 * _CDEG, _F), lambda i: (i, 0))],
        out_specs=pl.BlockSpec((8, _F), lambda i: (i, 0)),
        out_shape=jax.ShapeDtypeStruct((_M, _F), jnp.float32),
    )(r_char)


def _prep_body(g_ref, w_ref, b_ref, wf_ref, wc_ref, a_ref,
               u_ref, v_ref, wtp_ref, wbp_ref, wa_ref, cf_ref):
    w = w_ref[...]
    b = b_ref[...]
    c = jnp.dot(g_ref[...], w, preferred_element_type=jnp.float32) + 64.0 * b
    u_ref[...] = jnp.dot(c, wc_ref[:_F, :], preferred_element_type=jnp.float32)
    v_ref[...] = jnp.dot(c, wc_ref[_F:, :], preferred_element_type=jnp.float32)
    wft = wf_ref[:_F, :]
    wfb = wf_ref[_F:, :]
    wtp_ref[...] = jnp.dot(w, wft, preferred_element_type=jnp.float32)
    wbp_ref[...] = jnp.dot(w, wfb, preferred_element_type=jnp.float32)
    wa_ref[...] = jnp.dot(w, a_ref[...], preferred_element_type=jnp.float32)
    cf_ref[...] = jnp.dot(b, wft + wfb, preferred_element_type=jnp.float32)


def _tc_prep(g, w, b2, wf, wc, a):
    shapes = [
        jax.ShapeDtypeStruct((_M, _F), jnp.float32),
        jax.ShapeDtypeStruct((_M, _F), jnp.float32),
        jax.ShapeDtypeStruct((_F, _F), jnp.float32),
        jax.ShapeDtypeStruct((_F, _F), jnp.float32),
        jax.ShapeDtypeStruct((_F, 1), jnp.float32),
        jax.ShapeDtypeStruct((1, _F), jnp.float32),
    ]
    return pl.pallas_call(
        _prep_body,
        out_shape=shapes,
    )(g, w, b2, wf, wc, a)


_BB = 256


def _flow_body(r2_ref, fid_ref, wtp_ref, wbp_ref, wa_ref, cf_ref, fl_ref):
    r = r2_ref[...]
    wa = wa_ref[...][:, 0]
    vals = jnp.sum(r * wa[None, None, :], axis=2, keepdims=True)
    m = jnp.max(vals, axis=1, keepdims=True)
    p = jnp.exp(vals - m)
    attn = p / jnp.sum(p, axis=1, keepdims=True)
    xagg = jnp.sum(r * attn, axis=1)
    fl = jnp.tanh(
        jnp.dot(fid_ref[...], wtp_ref[...], preferred_element_type=jnp.float32)
        + jnp.dot(xagg, wbp_ref[...], preferred_element_type=jnp.float32)
        + cf_ref[...]
    )
    fl_ref[...] = fl


def _tc_flow(r2, r1, wtp, wbp, wa, cf):
    off = (_CDEG * _M) // _BB
    return pl.pallas_call(
        _flow_body,
        grid=(2 * _B // _BB,),
        in_specs=[
            pl.BlockSpec((_BB, _DEG, _F), lambda i: (i, 0, 0)),
            pl.BlockSpec((_BB, _F), lambda i: (i + off, 0)),
            pl.BlockSpec((_F, _F), lambda i: (0, 0)),
            pl.BlockSpec((_F, _F), lambda i: (0, 0)),
            pl.BlockSpec((_F, 1), lambda i: (0, 0)),
            pl.BlockSpec((1, _F), lambda i: (0, 0)),
        ],
        out_specs=pl.BlockSpec((_BB, _F), lambda i: (i, 0)),
        out_shape=jax.ShapeDtypeStruct((2 * _B, _F), jnp.float32),
    )(r2, r1, wtp, wbp, wa, cf)


_DB = 512


def _decode_body(fla_ref, fln_ref, uua_ref, uun_ref, vva_ref, vvn_ref, o_ref):
    fla = fla_ref[...]
    fln = fln_ref[...]
    uua = uua_ref[...]
    uun = uun_ref[...]
    vva = vva_ref[...]
    vvn = vvn_ref[...]

    def score(fl, u, v, k):
        cl = jax.nn.sigmoid(u + v)
        s = jnp.sum(fl * cl, axis=1, keepdims=True)
        o_ref[:, k : k + 1] = jax.nn.sigmoid(s)

    score(fla, uua, vva, 0)
    score(fla, uun, vva, 1)
    score(fln, uun, vvn, 2)
    score(fln, uua, vvn, 3)


def _tc_decode(fl, uu, vv):
    half = _B // _DB
    return pl.pallas_call(
        _decode_body,
        grid=(half,),
        in_specs=[
            pl.BlockSpec((_DB, _F), lambda i: (i, 0)),
            pl.BlockSpec((_DB, _F), lambda i: (i + half, 0)),
            pl.BlockSpec((_DB, _F), lambda i: (i, 0)),
            pl.BlockSpec((_DB, _F), lambda i: (i + half, 0)),
            pl.BlockSpec((_DB, _F), lambda i: (i, 0)),
            pl.BlockSpec((_DB, _F), lambda i: (i + half, 0)),
        ],
        out_specs=pl.BlockSpec((_DB, 4), lambda i: (i, 0)),
        out_shape=jax.ShapeDtypeStruct((_B, 4), jnp.float32),
    )(fl, fl, uu, uu, vv, vv)


def kernel(feature, flow_adj, flow_char_adj, item_id, category, PA_level,
           weight_emb, bias_emb, weight_character, a_attn, weight_flow):
    feature = feature.astype(jnp.float32)
    ids = item_id.T.reshape(-1).astype(jnp.int32)
    idx_a = jnp.concatenate(
        [flow_char_adj.reshape(-1).astype(jnp.int32), ids]
    ).reshape(1, -1)
    idx_b = ids.reshape(1, -1)

    fadj_pad = jnp.pad(flow_adj.astype(jnp.int32), ((0, 0), (0, 128 - _DEG)))
    r1, nb = _sc_gather_feat_nbrs(feature, fadj_pad, idx_a, idx_b)
    r2 = _sc_gather_rows(feature, nb[:, :_DEG].reshape(1, -1))

    g = _tc_segsum(r1[: _CDEG * _M])
    u_tab, v_tab, wtp, wbp, wa, cf = _tc_prep(
        g, weight_emb, bias_emb.reshape(1, _F), weight_flow, weight_character, a_attn
    )

    idx_u = jnp.concatenate(
        [category[:, 0], category[:, 1]]
    ).astype(jnp.int32).reshape(1, -1)
    idx_v = jnp.concatenate(
        [PA_level[:, 0], PA_level[:, 1]]
    ).astype(jnp.int32).reshape(1, -1)
    uu, vv = _sc_gather_uv(u_tab, v_tab, idx_u, idx_v)

    fl = _tc_flow(r2.reshape(2 * _B, _DEG, _F), r1, wtp, wbp, wa, cf)
    p = _tc_decode(fl, uu, vv)
    return (p[:, 0], p[:, 1], p[:, 2], p[:, 3])
